# baseline (speedup 1.0000x reference)
import jax
import jax.numpy as jnp
from jax.experimental import pallas as pl

_NE = 1024
_D = 64
_P = 512


def _body(x_ref, e_ref, o_ref):
    x = x_ref[...]
    e = e_ref[...]
    x2 = jnp.sum(x * x, axis=1, keepdims=True)
    e2 = jnp.sum(e * e, axis=1)
    mm = jax.lax.dot_general(x, e, (((1,), (1,)), ((), ())))
    dist = (x2 + e2[None, :]) - 2.0 * mm
    m = jnp.min(dist, axis=1, keepdims=True)
    jidx = jax.lax.broadcasted_iota(jnp.int32, (_P, _NE), 1)
    idx = jnp.min(jnp.where(dist == m, jidx, _NE), axis=1, keepdims=True)
    oh = (jidx == idx).astype(jnp.float32)
    q = jax.lax.dot_general(oh, e, (((1,), (0,)), ((), ())),
                            precision=jax.lax.Precision.HIGHEST)
    o_ref[...] = x + (q - x)


def kernel(inputs, embedding):
    x = jnp.transpose(inputs, (0, 2, 3, 1)).reshape(-1, _D)
    n = x.shape[0]
    out = pl.pallas_call(
        _body,
        grid=(n // _P,),
        in_specs=[pl.BlockSpec((_P, _D), lambda i: (i, 0)),
                  pl.BlockSpec((_NE, _D), lambda i: (0, 0))],
        out_specs=pl.BlockSpec((_P, _D), lambda i: (i, 0)),
        out_shape=jax.ShapeDtypeStruct((n, _D), jnp.float32),
    )(x, embedding)
    b, c, h, w = inputs.shape
    return out.reshape(b, h, w, c).transpose(0, 3, 1, 2)


# speedup vs baseline: 1.5715x; 1.5715x over previous
import jax
import jax.numpy as jnp
from jax.experimental import pallas as pl

_NE = 1024
_D = 64
_P = 512


def _body(x_ref, e_ref, o_ref):
    x = jnp.transpose(x_ref[0], (1, 0))
    e = e_ref[...]
    x2 = jnp.sum(x * x, axis=1, keepdims=True)
    e2 = jnp.sum(e * e, axis=1)
    mm = jax.lax.dot_general(x, e, (((1,), (1,)), ((), ())))
    dist = (x2 + e2[None, :]) - 2.0 * mm
    m = jnp.min(dist, axis=1, keepdims=True)
    jidx = jax.lax.broadcasted_iota(jnp.int32, (_P, _NE), 1)
    idx = jnp.min(jnp.where(dist == m, jidx, _NE), axis=1, keepdims=True)
    oh = (jidx == idx).astype(jnp.float32)
    q = jax.lax.dot_general(oh, e, (((1,), (0,)), ((), ())))
    o_ref[0] = jnp.transpose(x + (q - x), (1, 0))


def kernel(inputs, embedding):
    b, c, h, w = inputs.shape
    xf = inputs.reshape(b, c, h * w)
    npix = h * w
    out = pl.pallas_call(
        _body,
        grid=(b, npix // _P),
        in_specs=[pl.BlockSpec((1, c, _P), lambda i, j: (i, 0, j)),
                  pl.BlockSpec((_NE, _D), lambda i, j: (0, 0))],
        out_specs=pl.BlockSpec((1, c, _P), lambda i, j: (i, 0, j)),
        out_shape=jax.ShapeDtypeStruct((b, c, npix), jnp.float32),
    )(xf, embedding)
    return out.reshape(b, c, h, w)


# speedup vs baseline: 1.7240x; 1.0970x over previous
import jax
import jax.numpy as jnp
from jax.experimental import pallas as pl
from jax.experimental.pallas import tpu as pltpu

_NE = 1024
_D = 64
_P = 1024


def _body(x_ref, e_ref, o_ref, e2_ref):
    e = e_ref[...]

    @pl.when((pl.program_id(0) == 0) & (pl.program_id(1) == 0))
    def _init():
        e2_ref[0, :] = jnp.sum(e * e, axis=1)

    x = jnp.transpose(x_ref[0], (1, 0))
    x2 = jnp.sum(x * x, axis=1, keepdims=True)
    e2 = e2_ref[...]
    mm = jax.lax.dot_general(x, e, (((1,), (1,)), ((), ())))
    dist = (x2 + e2) - 2.0 * mm
    m = jnp.min(dist, axis=1, keepdims=True)
    jidx = jax.lax.broadcasted_iota(jnp.int32, (_P, _NE), 1)
    idx = jnp.min(jnp.where(dist == m, jidx, _NE), axis=1, keepdims=True)
    oh = (jidx == idx).astype(jnp.float32)
    q = jax.lax.dot_general(oh, e, (((1,), (0,)), ((), ())))
    o_ref[0] = jnp.transpose(x + (q - x), (1, 0))


def kernel(inputs, embedding):
    b, c, h, w = inputs.shape
    xf = inputs.reshape(b, c, h * w)
    npix = h * w
    out = pl.pallas_call(
        _body,
        grid=(b, npix // _P),
        in_specs=[pl.BlockSpec((1, c, _P), lambda i, j: (i, 0, j)),
                  pl.BlockSpec((_NE, _D), lambda i, j: (0, 0))],
        out_specs=pl.BlockSpec((1, c, _P), lambda i, j: (i, 0, j)),
        out_shape=jax.ShapeDtypeStruct((b, c, npix), jnp.float32),
        scratch_shapes=[pltpu.VMEM((1, _NE), jnp.float32)],
    )(xf, embedding)
    return out.reshape(b, c, h, w)
